# Initial kernel scaffold; baseline (speedup 1.0000x reference)
import jax
import jax.numpy as jnp
from jax.experimental import pallas as pl


def kernel(x, edge_index, batch, W1, as1, ad1, b1, W2, as2, ad2, b2, W3, as3, ad3, b3, W4, as4, ad4, b4, lw1, lb1, lw2, lb2):
    raise NotImplementedError("write your pallas kernel here")


# speedup vs baseline: 1.0010x; 1.0010x over previous
import jax
import jax.numpy as jnp
from jax.experimental import pallas as pl
from jax.experimental.pallas import tpu as pltpu

N_GRAPHS = 64


def _gat_conv(x, edge_index, W, att_src, att_dst, bias):
    N = x.shape[0]
    heads = att_src.shape[1]
    out_ch = att_src.shape[2]
    loop = jnp.arange(N, dtype=edge_index.dtype)
    src = jnp.concatenate([edge_index[0], loop])
    dst = jnp.concatenate([edge_index[1], loop])
    h = (x @ W).reshape(N, heads, out_ch)
    a_src = jnp.sum(h * att_src, axis=-1)
    a_dst = jnp.sum(h * att_dst, axis=-1)
    e = jax.nn.leaky_relu(a_src[src] + a_dst[dst], 0.2)
    e_max = jax.ops.segment_max(e, dst, num_segments=N)
    e_max = jnp.where(jnp.isfinite(e_max), e_max, 0.0)
    ex = jnp.exp(e - e_max[dst])
    denom = jax.ops.segment_sum(ex, dst, num_segments=N)
    alpha = ex / (denom[dst] + 1e-16)
    out = jax.ops.segment_sum(h[src] * alpha[..., None], dst, num_segments=N)
    return out.reshape(N, heads * out_ch) + bias


def _pool_mlp_body(h_ref, batch_ref, lw1_ref, lb1_ref, lw2_ref, lb2_ref, out_ref):
    h = h_ref[...]
    b = batch_ref[...]
    gids = jax.lax.broadcasted_iota(jnp.int32, (N_GRAPHS, h.shape[0]), 0)
    P = (b == gids).astype(jnp.float32)
    s = jnp.dot(P, h, preferred_element_type=jnp.float32,
                precision=jax.lax.Precision.HIGHEST)
    cnt = jnp.sum(P, axis=1, keepdims=True)
    m = s / jnp.maximum(cnt, 1.0)
    g = jnp.dot(m, lw1_ref[...], precision=jax.lax.Precision.HIGHEST) + lb1_ref[...]
    g = jnp.where(g > 0, g, jnp.exp(g) - 1.0)
    out_ref[...] = jnp.dot(g, lw2_ref[...], precision=jax.lax.Precision.HIGHEST) + lb2_ref[...]


def _pool_mlp(h, batch, lw1, lb1, lw2, lb2):
    return pl.pallas_call(
        _pool_mlp_body,
        out_shape=jax.ShapeDtypeStruct((N_GRAPHS, lw2.shape[1]), jnp.float32),
    )(h, batch.reshape(1, -1), lw1, lb1.reshape(1, -1), lw2, lb2.reshape(1, -1))


def kernel(x, edge_index, batch, W1, as1, ad1, b1, W2, as2, ad2, b2, W3, as3,
           ad3, b3, W4, as4, ad4, b4, lw1, lb1, lw2, lb2):
    h = jax.nn.elu(_gat_conv(x, edge_index, W1, as1, ad1, b1))
    h = jax.nn.elu(_gat_conv(h, edge_index, W2, as2, ad2, b2))
    h = jax.nn.elu(_gat_conv(h, edge_index, W3, as3, ad3, b3))
    h = _gat_conv(h, edge_index, W4, as4, ad4, b4)
    return _pool_mlp(h, batch, lw1, lb1, lw2, lb2)


# speedup vs baseline: 44.4027x; 44.3593x over previous
import jax
import jax.numpy as jnp
from jax import lax
from jax.experimental import pallas as pl
from jax.experimental.pallas import tpu as pltpu
from jax.experimental.pallas import tpu_sc as plsc

N_NODES = 10000
N_EDGES = 320000
N_GRAPHS = 64
HEADS = 8

NW = 32
NPW = 320
PADN = NW * NPW
CH = 128
EP = N_EDGES + N_NODES
PADE = ((EP + CH + 32 + 63) // 64) * 64
RPLEN = PADN + 32

_HIGH = jax.lax.Precision.HIGHEST
_NEG = -3.0e38


def _index_prep(edge_index):
    loop = jnp.arange(N_NODES, dtype=jnp.int32)
    src = jnp.concatenate([edge_index[0].astype(jnp.int32), loop])
    dst = jnp.concatenate([edge_index[1].astype(jnp.int32), loop])
    order = jnp.argsort(dst)
    dst_s = dst[order]
    src_s = src[order]
    srcs = jnp.zeros((PADE,), jnp.int32).at[:EP].set(src_s)
    dsts = jnp.zeros((PADE,), jnp.int32).at[:EP].set(dst_s)
    rp = jnp.searchsorted(dst_s, jnp.arange(RPLEN, dtype=jnp.int32),
                          side="left").astype(jnp.int32)
    return srcs, dsts, rp


def _tc_layer_body(x_ref, w_ref, a_ref, hx_ref, a2_ref):
    h = jnp.dot(x_ref[...], w_ref[...], preferred_element_type=jnp.float32,
                precision=_HIGH)
    a2 = jnp.dot(h, a_ref[...], preferred_element_type=jnp.float32,
                 precision=_HIGH)
    pad = jnp.zeros((h.shape[0], 112), jnp.float32)
    hx_ref[...] = jnp.concatenate([h, a2, pad], axis=1)
    a2_ref[...] = a2


def _tc_layer(x_pad, W, A):
    d_in, d_out = W.shape
    bn = 1280
    grid = (PADN // bn,)
    return pl.pallas_call(
        _tc_layer_body,
        grid=grid,
        in_specs=[
            pl.BlockSpec((bn, d_in), lambda i: (i, 0)),
            pl.BlockSpec((d_in, d_out), lambda i: (0, 0)),
            pl.BlockSpec((d_out, 16), lambda i: (0, 0)),
        ],
        out_specs=[
            pl.BlockSpec((bn, d_out + 128), lambda i: (i, 0)),
            pl.BlockSpec((bn, 16), lambda i: (i, 0)),
        ],
        out_shape=[
            jax.ShapeDtypeStruct((PADN, d_out + 128), jnp.float32),
            jax.ShapeDtypeStruct((PADN, 16), jnp.float32),
        ],
    )(x_pad, W, A)


def _att_mat(att_src, att_dst):
    oc = att_src.shape[2]
    d_out = HEADS * oc
    A = jnp.zeros((d_out, 16), jnp.float32)
    for k in range(HEADS):
        A = A.at[k * oc:(k + 1) * oc, k].set(att_src[0, k])
        A = A.at[k * oc:(k + 1) * oc, 8 + k].set(att_dst[0, k])
    return A


def _make_sc_layer(d_out, do_elu):
    dx = d_out + 128
    nv = d_out // 16
    ocp = nv // HEADS
    mesh = plsc.VectorSubcoreMesh(core_axis_name="c", subcore_axis_name="s")

    def body(hx_hbm, a2f_hbm, srcs_hbm, dsts_hbm, rp_hbm, bias_hbm, out_hbm,
             sidx, sdst, gidx, hxr, ebuf, a2own, rot, orow, biasv, rpv, sem):
        cid = lax.axis_index("c")
        sid = lax.axis_index("s")
        wid = sid * 2 + cid
        n0 = pl.multiple_of(wid * NPW, NPW)
        nn = jnp.minimum(N_NODES - n0, NPW)

        pltpu.sync_copy(rp_hbm.at[pl.ds(n0, NPW + 32)], rpv)
        pltpu.sync_copy(a2f_hbm.at[pl.ds(pl.multiple_of(n0 * 16, 128),
                                         NPW * 16)], a2own)
        pltpu.sync_copy(bias_hbm, biasv)

        lanei = lax.iota(jnp.int32, 16)
        e0 = rpv[pl.ds(0, 16)][0]
        e1 = rpv[pl.ds(nn, 16)][0]
        nch = lax.div(e1 - e0 + (CH - 1), jnp.int32(CH))

        m_init = jnp.full((16,), _NEG, jnp.float32)
        r_init = jnp.zeros((16,), jnp.float32)
        acc_init = tuple(jnp.zeros((16,), jnp.float32) for _ in range(nv))
        carry0 = (m_init, r_init) + acc_init

        @pl.loop(0, nch, init_carry=carry0)
        def _chunks(ci, carry):
            cs = e0 + ci * CH
            ce = jnp.minimum(cs + CH, e1)
            L = ce - cs
            off = lax.rem(cs, 8)
            base = pl.multiple_of(cs - off, 8)
            pltpu.sync_copy(srcs_hbm.at[pl.ds(base, CH + 32)], sidx)
            pltpu.sync_copy(dsts_hbm.at[pl.ds(base, CH + 32)], sdst)
            for g in range(CH // 16):
                v = sidx[pl.ds(off + g * 16, 16)]
                ok = (g * 16 + lanei) < L
                gidx[pl.ds(g * 16, 16)] = jnp.where(ok, v, 0)
            pltpu.async_copy(hx_hbm.at[gidx], hxr, sem).wait()

            na = sdst[pl.ds(off, 16)][0] - n0
            nb = sdst[pl.ds(off + L - 1, 16)][0] - n0

            @pl.loop(na, nb + 1, init_carry=carry)
            def _nodes(nl, c):
                m, r = c[0], c[1]
                acc = c[2:]
                s_n = rpv[pl.ds(nl, 16)][0]
                t_n = rpv[pl.ds(nl + 1, 16)][0]
                lo = jnp.maximum(s_n, cs) - cs
                hi = jnp.minimum(t_n, ce) - cs

                a2n = a2own[pl.ds(nl * 16, 16)]
                rot[pl.ds(0, 16)] = a2n
                rot[pl.ds(16, 16)] = a2n
                adst = rot[pl.ds(8, 16)]

                def f_a(j, mc):
                    av = hxr[j, pl.ds(d_out, 16)]
                    e = av + adst
                    e = jnp.where(e > 0, e, 0.2 * e)
                    ebuf[pl.ds(j * 16, 16)] = e
                    return jnp.maximum(mc, e)
                mc = lax.fori_loop(lo, hi, f_a, m_init)

                m2 = jnp.maximum(m, mc)
                scale = jnp.exp(m - m2)
                rs = r * scale
                accs = tuple(a * scale for a in acc)

                def f_b(j, cb):
                    rb = cb[0]
                    ab = cb[1:]
                    ev = ebuf[pl.ds(j * 16, 16)]
                    w = jnp.exp(ev - m2)
                    new = []
                    for k in range(HEADS):
                        wk = jnp.full((16,), w[k], jnp.float32)
                        for q in range(ocp):
                            vi = k * ocp + q
                            hv = hxr[j, pl.ds(vi * 16, 16)]
                            new.append(ab[vi] + wk * hv)
                    return (rb + w,) + tuple(new)

                res = lax.fori_loop(lo, hi, f_b, (rs,) + accs)
                r3 = res[0]
                acc3 = res[1:]

                done = t_n <= ce

                @pl.when(done)
                def _():
                    rinv = 1.0 / (r3 + 1e-16)
                    for k in range(HEADS):
                        rk = jnp.full((16,), rinv[k], jnp.float32)
                        for q in range(ocp):
                            vi = k * ocp + q
                            v = acc3[vi] * rk + biasv[pl.ds(vi * 16, 16)]
                            if do_elu:
                                v = jnp.where(v > 0, v, jnp.exp(v) - 1.0)
                            orow[pl.ds(vi * 16, 16)] = v
                    oof = pl.multiple_of((n0 + nl) * d_out, 128)
                    pltpu.sync_copy(orow, out_hbm.at[pl.ds(oof, d_out)])

                m4 = jnp.where(done, m_init, m2)
                r4 = jnp.where(done, r_init, r3)
                acc4 = tuple(jnp.where(done, jnp.zeros((16,), jnp.float32), a)
                             for a in acc3)
                return (m4, r4) + acc4

            return _nodes

    kern = pl.kernel(
        body,
        out_type=jax.ShapeDtypeStruct((PADN * d_out,), jnp.float32),
        mesh=mesh,
        scratch_types=[
            pltpu.VMEM((CH + 32,), jnp.int32),
            pltpu.VMEM((CH + 32,), jnp.int32),
            pltpu.VMEM((CH,), jnp.int32),
            pltpu.VMEM((CH, dx), jnp.float32),
            pltpu.VMEM((CH * 16,), jnp.float32),
            pltpu.VMEM((NPW * 16,), jnp.float32),
            pltpu.VMEM((32,), jnp.float32),
            pltpu.VMEM((d_out,), jnp.float32),
            pltpu.VMEM((d_out,), jnp.float32),
            pltpu.VMEM((NPW + 32,), jnp.int32),
            pltpu.SemaphoreType.DMA,
        ],
    )
    return kern


def _pool_mlp_body(h_ref, batch_ref, lw1_ref, lb1_ref, lw2_ref, lb2_ref,
                   out_ref):
    h = h_ref[...]
    b = batch_ref[...]
    gids = jax.lax.broadcasted_iota(jnp.int32, (N_GRAPHS, h.shape[0]), 0)
    P = (b == gids).astype(jnp.float32)
    s = jnp.dot(P, h, preferred_element_type=jnp.float32, precision=_HIGH)
    cnt = jnp.sum(P, axis=1, keepdims=True)
    m = s / jnp.maximum(cnt, 1.0)
    g = jnp.dot(m, lw1_ref[...], precision=_HIGH) + lb1_ref[...]
    g = jnp.where(g > 0, g, jnp.exp(g) - 1.0)
    out_ref[...] = jnp.dot(g, lw2_ref[...], precision=_HIGH) + lb2_ref[...]


def _pool_mlp(h, batch, lw1, lb1, lw2, lb2):
    return pl.pallas_call(
        _pool_mlp_body,
        out_shape=jax.ShapeDtypeStruct((N_GRAPHS, lw2.shape[1]), jnp.float32),
    )(h, batch.reshape(1, -1), lw1, lb1.reshape(1, -1), lw2,
      lb2.reshape(1, -1))


def kernel(x, edge_index, batch, W1, as1, ad1, b1, W2, as2, ad2, b2, W3, as3,
           ad3, b3, W4, as4, ad4, b4, lw1, lb1, lw2, lb2):
    srcs, dsts, rp = _index_prep(edge_index)
    x_pad = jnp.zeros((PADN, x.shape[1]), jnp.float32).at[:N_NODES].set(x)

    layers = [
        (W1, as1, ad1, b1, True),
        (W2, as2, ad2, b2, True),
        (W3, as3, ad3, b3, True),
        (W4, as4, ad4, b4, False),
    ]
    h_cur = x_pad
    for W, a_s, a_d, bias, do_elu in layers:
        d_out = W.shape[1]
        A = _att_mat(a_s, a_d)
        hx, a2 = _tc_layer(h_cur, W, A)
        a2f = a2.reshape(-1)
        sc = _make_sc_layer(d_out, do_elu)
        outf = sc(hx, a2f, srcs, dsts, rp, bias)
        h_cur = outf.reshape(PADN, d_out)

    return _pool_mlp(h_cur[:N_NODES], batch, lw1, lb1, lw2, lb2)
